# Initial kernel scaffold; baseline (speedup 1.0000x reference)
import jax
import jax.numpy as jnp
from jax.experimental import pallas as pl


def kernel(a_hat, x_b, w1, b1, w2, b2, wih, bih, bhh, wout, bout):
    raise NotImplementedError("write your pallas kernel here")


# speedup vs baseline: 16.8606x; 16.8606x over previous
import jax
import jax.numpy as jnp
from jax.experimental import pallas as pl
from jax.experimental.pallas import tpu as pltpu

N = 16
C = 4
H1 = 32
H2 = 16
HL = 8
BLK = 4096


def _body(x_ref, m1_ref, m2_ref, m3_ref, r_ref, aux_ref, o_ref):
    f32 = jnp.float32
    bf16 = jnp.bfloat16
    x = x_ref[...].astype(bf16)
    z1 = jnp.dot(x, m1_ref[...], preferred_element_type=f32)
    h1 = jnp.maximum(z1 + aux_ref[0:1, :], 0.0).astype(bf16)
    z2 = jnp.dot(h1, m2_ref[...], preferred_element_type=f32)
    h2 = jnp.maximum(z2 + aux_ref[1:2, 0:256], 0.0).astype(bf16)
    g = jnp.dot(h2, m3_ref[...], preferred_element_type=f32)
    g = g + aux_ref[2:3, 0:384]
    i_g = jax.nn.sigmoid(g[:, 0:128])
    g_g = jnp.tanh(g[:, 128:256])
    o_g = jax.nn.sigmoid(g[:, 256:384])
    h = (o_g * jnp.tanh(i_g * g_g)).astype(bf16)
    y = jnp.dot(h, r_ref[...], preferred_element_type=f32)
    o_ref[...] = y + aux_ref[3:4, 0:16]


def kernel(a_hat, x_b, w1, b1, w2, b2, wih, bih, bhh, wout, bout):
    f32 = jnp.float32
    bf16 = jnp.bfloat16
    B = x_b.shape[0]
    xv = x_b.reshape(B, N * C)
    pad = (-B) % BLK
    if pad:
        xv = jnp.concatenate([xv, jnp.zeros((pad, N * C), xv.dtype)], axis=0)
    nb = (B + pad) // BLK

    eye = jnp.eye(N, dtype=f32)
    m1 = jnp.einsum('nm,cj->mcnj', a_hat, w1).reshape(N * C, N * H1)
    m2 = jnp.einsum('nm,cj->mcnj', a_hat, w2).reshape(N * H1, N * H2)
    wg = jnp.concatenate([wih[:, 0:HL], wih[:, 2 * HL:4 * HL]], axis=1)
    m3 = jnp.einsum('ck,mn->mckn', wg, eye).reshape(N * H2, N * 3 * HL)
    r = jnp.einsum('j,nm->jnm', wout[:, 0], eye).reshape(N * HL, N)

    b1v = jnp.tile(b1[0], N)
    b2v = jnp.tile(b2[0], N)
    bg = (bih + bhh)[0]
    bgv = jnp.repeat(jnp.concatenate([bg[0:HL], bg[2 * HL:4 * HL]]), N)
    aux = jnp.zeros((8, N * H1), f32)
    aux = aux.at[0, :].set(b1v)
    aux = aux.at[1, 0:N * H2].set(b2v)
    aux = aux.at[2, 0:N * 3 * HL].set(bgv)
    aux = aux.at[3, :].set(bout[0, 0])

    out = pl.pallas_call(
        _body,
        out_shape=jax.ShapeDtypeStruct((B + pad, N), f32),
        grid=(nb,),
        in_specs=[
            pl.BlockSpec((BLK, N * C), lambda i: (i, 0)),
            pl.BlockSpec((N * C, N * H1), lambda i: (0, 0)),
            pl.BlockSpec((N * H1, N * H2), lambda i: (0, 0)),
            pl.BlockSpec((N * H2, N * 3 * HL), lambda i: (0, 0)),
            pl.BlockSpec((N * HL, N), lambda i: (0, 0)),
            pl.BlockSpec((8, N * H1), lambda i: (0, 0)),
        ],
        out_specs=pl.BlockSpec((BLK, N), lambda i: (i, 0)),
        compiler_params=pltpu.CompilerParams(
            dimension_semantics=("parallel",)),
    )(xv, m1.astype(bf16), m2.astype(bf16), m3.astype(bf16),
      r.astype(bf16), aux)
    return out[:B]
